# baseline (speedup 1.0000x reference)
import functools

import jax
import jax.numpy as jnp
from jax import lax
from jax.experimental import pallas as pl
from jax.experimental.pallas import tpu as pltpu
from jax.experimental.pallas import tpu_sc as plsc

N_NODES = 10000
N_EDGES = 320000
D = 128

NC = 2
NS = 16
HALF = N_NODES // NC
EPT = N_EDGES // NS
K = 128
BLK8 = 8
NCHUNK = 160
NBLK = NCHUNK // BLK8
EPT_PAD = NCHUNK * K
ACC_N = 5120
ACC2_N = 2 * ACC_N
ACC_RPT = ACC_N // NS
DEG_N = 10112
DEG_RPT = DEG_N // NS
DEG_W = 16
DUMMY_DST = N_NODES


def _sc_mesh():
    return plsc.VectorSubcoreMesh(core_axis_name="c", subcore_axis_name="s",
                                  num_cores=NC, num_subcores=NS)


def _stage_remap(d8, row, dstj, lo, hi, base, dummy, off):
    for kk in range(K // 16):
        d = d8[row, pl.ds(kk * 16, 16)]
        ok = (d >= lo) & (d < hi)
        dstj[pl.ds(kk * 16, 16)] = jnp.where(ok, d - base + off, dummy + off)


def _sc_degree(dst4, zeros_acc, ones_blk):
    out_ty = jax.ShapeDtypeStruct((ACC2_N, D), jnp.float32)

    @functools.partial(
        pl.kernel, mesh=_sc_mesh(), out_type=out_ty,
        scratch_types=[
            pltpu.VMEM((BLK8, K), jnp.int32),
            pltpu.VMEM((K,), jnp.int32),
            pltpu.VMEM((K, D), jnp.float32),
            pltpu.VMEM_SHARED((ACC2_N, D), jnp.float32),
        ])
    def kern(dst_h, zero_h, ones_h, out_h, dst8, dstj, ones_v, acc):
        c = lax.axis_index("c")
        s = lax.axis_index("s")
        lo = c * HALF
        off = c * ACC_N
        sl = pl.ds(off + s * ACC_RPT, ACC_RPT)

        pltpu.sync_copy(zero_h.at[pl.ds(s * ACC_RPT, ACC_RPT)], acc.at[sl])
        pltpu.sync_copy(ones_h, ones_v)
        pltpu.sync_copy(dst_h.at[s, 0], dst8)
        plsc.subcore_barrier()

        @pl.loop(0, NCHUNK)
        def _(j):
            r0 = lax.rem(j, BLK8)
            _stage_remap(dst8, r0, dstj, lo, lo + HALF, lo, HALF, off)
            pltpu.sync_copy(ones_v, acc.at[dstj], add=True)

            @pl.when((j + 1 < NCHUNK) & (r0 + 1 >= BLK8))
            def _():
                pltpu.sync_copy(dst_h.at[s, lax.div(j + 1, BLK8)], dst8)

        plsc.subcore_barrier()
        pltpu.sync_copy(acc.at[sl], out_h.at[sl])

    return kern(dst4, zeros_acc, ones_blk)


def _sc_propagate(table, src4, dst4, zeros_acc):
    out_ty = jax.ShapeDtypeStruct((ACC2_N, D), jnp.float32)

    @functools.partial(
        pl.kernel, mesh=_sc_mesh(), out_type=out_ty,
        scratch_types=[
            pltpu.VMEM((BLK8, K), jnp.int32),
            pltpu.VMEM((BLK8, K), jnp.int32),
            pltpu.VMEM((K,), jnp.int32),
            pltpu.VMEM((K,), jnp.int32),
            pltpu.VMEM((K, D), jnp.float32),
            pltpu.VMEM((K, D), jnp.float32),
            pltpu.VMEM_SHARED((ACC2_N, D), jnp.float32),
            pltpu.SemaphoreType.DMA,
            pltpu.SemaphoreType.DMA,
            pltpu.SemaphoreType.DMA,
            pltpu.SemaphoreType.DMA,
        ])
    def kern(table_h, src_h, dst_h, zero_h, out_h,
             src8, dst8, dstja, dstjb, bufa, bufb,
             acc, semga, semgb, semsa, semsb):
        c = lax.axis_index("c")
        s = lax.axis_index("s")
        lo = c * HALF
        off = c * ACC_N
        sl = pl.ds(off + s * ACC_RPT, ACC_RPT)

        pltpu.sync_copy(zero_h.at[pl.ds(s * ACC_RPT, ACC_RPT)], acc.at[sl])
        plsc.subcore_barrier()

        def scatter_start(buf, dstj, sem):
            pltpu.async_copy(buf, acc.at[dstj], sem, add=True)

        def scatter_wait(buf, dstj, sem):
            pltpu.make_async_copy(buf, acc.at[dstj], sem).wait()

        pltpu.sync_copy(src_h.at[s, 0], src8)
        pltpu.sync_copy(dst_h.at[s, 0], dst8)
        pltpu.async_copy(table_h.at[src8.at[0]], bufa, semga)

        @pl.loop(0, NCHUNK, step=2)
        def _(j):
            r0 = lax.rem(j, BLK8)

            @pl.when(j > 0)
            def _():
                scatter_wait(bufb, dstjb, semsb)

            pltpu.async_copy(table_h.at[src8.at[r0 + 1]], bufb, semgb)

            pltpu.make_async_copy(table_h.at[src8.at[0]], bufa, semga).wait()
            _stage_remap(dst8, r0, dstja, lo, lo + HALF, lo, HALF, off)
            scatter_start(bufa, dstja, semsa)
            _stage_remap(dst8, r0 + 1, dstjb, lo, lo + HALF, lo, HALF, off)

            pltpu.make_async_copy(table_h.at[src8.at[0]], bufb, semgb).wait()

            @pl.when((j + 2 < NCHUNK) & (r0 + 2 >= BLK8))
            def _():
                blk = lax.div(j + 2, BLK8)
                pltpu.sync_copy(src_h.at[s, blk], src8)
                pltpu.sync_copy(dst_h.at[s, blk], dst8)

            @pl.when(j + 2 < NCHUNK)
            def _():
                scatter_wait(bufa, dstja, semsa)
                pltpu.async_copy(table_h.at[src8.at[lax.rem(j + 2, BLK8)]],
                                 bufa, semga)

            scatter_start(bufb, dstjb, semsb)

        scatter_wait(bufa, dstja, semsa)
        scatter_wait(bufb, dstjb, semsb)
        plsc.subcore_barrier()
        pltpu.sync_copy(acc.at[sl], out_h.at[sl])

    return kern(table, src4, dst4, zeros_acc)


_BLK = 1000


def _tc_pre_body(deg_r, x_r, dis_r, xs_r):
    deg = deg_r[:, 0:1] + 1.0
    dis = lax.rsqrt(deg)
    disb = jnp.broadcast_to(dis, (_BLK, D))
    dis_r[...] = disb
    xs_r[...] = disb * x_r[...]


def _tc_pre(deg, x):
    return pl.pallas_call(
        _tc_pre_body,
        grid=(N_NODES // _BLK,),
        in_specs=[
            pl.BlockSpec((_BLK, D), lambda i: (i, 0)),
            pl.BlockSpec((_BLK, D), lambda i: (i, 0)),
        ],
        out_specs=[
            pl.BlockSpec((_BLK, D), lambda i: (i, 0)),
            pl.BlockSpec((_BLK, D), lambda i: (i, 0)),
        ],
        out_shape=[
            jax.ShapeDtypeStruct((N_NODES, D), jnp.float32),
            jax.ShapeDtypeStruct((N_NODES, D), jnp.float32),
        ],
    )(deg, x)


def _tc_mid_body(pa_r, xs_r, dis_r, w1_r, b1_r, w2_r, ts_r):
    p1 = dis_r[...] * (pa_r[...] + xs_r[...])
    h = jnp.dot(p1, w1_r[...], preferred_element_type=jnp.float32,
                precision=lax.Precision.HIGHEST) + b1_r[...]
    h = jnp.maximum(h, 0.0)
    t = jnp.dot(h, w2_r[...], preferred_element_type=jnp.float32,
                precision=lax.Precision.HIGHEST)
    ts_r[...] = dis_r[...] * t


def _tc_mid(pa, xs, dis, W1, b1, W2):
    d_hid = W1.shape[1]
    return pl.pallas_call(
        _tc_mid_body,
        grid=(N_NODES // _BLK,),
        in_specs=[
            pl.BlockSpec((_BLK, D), lambda i: (i, 0)),
            pl.BlockSpec((_BLK, D), lambda i: (i, 0)),
            pl.BlockSpec((_BLK, D), lambda i: (i, 0)),
            pl.BlockSpec((D, d_hid), lambda i: (0, 0)),
            pl.BlockSpec((1, d_hid), lambda i: (0, 0)),
            pl.BlockSpec((d_hid, D), lambda i: (0, 0)),
        ],
        out_specs=pl.BlockSpec((_BLK, D), lambda i: (i, 0)),
        out_shape=jax.ShapeDtypeStruct((N_NODES, D), jnp.float32),
    )(pa, xs, dis, W1, b1.reshape(1, d_hid), W2)


def _tc_post_body(pb_r, ts_r, dis_r, b2_r, out_r):
    out_r[...] = dis_r[...] * (pb_r[...] + ts_r[...]) + b2_r[...]


def _tc_post(pb, ts, dis, b2):
    return pl.pallas_call(
        _tc_post_body,
        grid=(N_NODES // _BLK,),
        in_specs=[
            pl.BlockSpec((_BLK, D), lambda i: (i, 0)),
            pl.BlockSpec((_BLK, D), lambda i: (i, 0)),
            pl.BlockSpec((_BLK, D), lambda i: (i, 0)),
            pl.BlockSpec((1, D), lambda i: (0, 0)),
        ],
        out_specs=pl.BlockSpec((_BLK, D), lambda i: (i, 0)),
        out_shape=jax.ShapeDtypeStruct((N_NODES, D), jnp.float32),
    )(pb, ts, dis, b2.reshape(1, D))


def _regions_to_full(arr):
    return jnp.concatenate([arr[:HALF], arr[ACC_N:ACC_N + HALF]], axis=0)


def kernel(x, edge_index, W1, b1, W2, b2):
    src = edge_index[0].astype(jnp.int32).reshape(NS, EPT)
    dst = edge_index[1].astype(jnp.int32).reshape(NS, EPT)
    pad = EPT_PAD - EPT
    src4 = jnp.pad(src, ((0, 0), (0, pad))).reshape(NS, NBLK, BLK8, K)
    dst4 = jnp.pad(dst, ((0, 0), (0, pad)),
                   constant_values=DUMMY_DST).reshape(NS, NBLK, BLK8, K)

    zeros_acc = jnp.zeros((ACC_N, D), jnp.float32)
    ones_blk = jnp.ones((K, D), jnp.float32)

    dg = _sc_degree(dst4, zeros_acc, ones_blk)
    dis, xs = _tc_pre(_regions_to_full(dg), x)
    pa = _sc_propagate(xs, src4, dst4, zeros_acc)
    ts = _tc_mid(_regions_to_full(pa), xs, dis, W1, b1, W2)
    pb = _sc_propagate(ts, src4, dst4, zeros_acc)
    return _tc_post(_regions_to_full(pb), ts, dis, b2)


# speedup vs baseline: 1.4736x; 1.4736x over previous
import functools

import jax
import jax.numpy as jnp
from jax import lax
from jax.experimental import pallas as pl
from jax.experimental.pallas import tpu as pltpu
from jax.experimental.pallas import tpu_sc as plsc

N_NODES = 10000
N_EDGES = 320000
D = 128

NC = 2
NS = 16
HALF = N_NODES // NC
EPT = N_EDGES // NS
K = 128
NCHUNK = 158
EPT_PAD = NCHUNK * K
ACC_N = 5120
ACC_RPT = ACC_N // NS
DUMMY_DST = N_NODES


def _sc_mesh():
    return plsc.VectorSubcoreMesh(core_axis_name="c", subcore_axis_name="s",
                                  num_cores=NC, num_subcores=NS)


def _stage_remap(dstr, dstj, lo, hi, base, dummy):
    for kk in range(K // 16):
        d = dstr[0, pl.ds(kk * 16, 16)]
        ok = (d >= lo) & (d < hi)
        dstj[pl.ds(kk * 16, 16)] = jnp.where(ok, d - base, dummy)


def _sc_degree(dst4, zeros_acc, ones_blk):
    out_ty = (jax.ShapeDtypeStruct((ACC_N, D), jnp.float32),) * 2

    @functools.partial(
        pl.kernel, mesh=_sc_mesh(), out_type=out_ty,
        scratch_types=[
            pltpu.VMEM((1, K), jnp.int32),
            pltpu.VMEM((K,), jnp.int32),
            pltpu.VMEM((K, D), jnp.float32),
            pltpu.VMEM_SHARED((ACC_N, D), jnp.float32),
            pltpu.VMEM_SHARED((ACC_N, D), jnp.float32),
        ])
    def kern(dst_h, zero_h, ones_h, outa_h, outb_h,
             dstr, dstj, ones_v, acc0, acc1):
        c = lax.axis_index("c")
        s = lax.axis_index("s")
        lo = c * HALF
        sl = pl.ds(s * ACC_RPT, ACC_RPT)

        @pl.when(c == 0)
        def _():
            pltpu.sync_copy(zero_h.at[sl], acc0.at[sl])

        @pl.when(c == 1)
        def _():
            pltpu.sync_copy(zero_h.at[sl], acc1.at[sl])

        pltpu.sync_copy(ones_h, ones_v)
        plsc.subcore_barrier()

        @pl.loop(0, NCHUNK)
        def _(j):
            pltpu.sync_copy(dst_h.at[s, j], dstr)
            _stage_remap(dstr, dstj, lo, lo + HALF, lo, HALF)

            @pl.when(c == 0)
            def _():
                pltpu.sync_copy(ones_v, acc0.at[dstj], add=True)

            @pl.when(c == 1)
            def _():
                pltpu.sync_copy(ones_v, acc1.at[dstj], add=True)

        plsc.subcore_barrier()

        @pl.when(c == 0)
        def _():
            pltpu.sync_copy(acc0.at[sl], outa_h.at[sl])

        @pl.when(c == 1)
        def _():
            pltpu.sync_copy(acc1.at[sl], outb_h.at[sl])

    return kern(dst4, zeros_acc, ones_blk)


def _sc_propagate(table, src4, dst4, zeros_acc):
    out_ty = (jax.ShapeDtypeStruct((ACC_N, D), jnp.float32),) * 2

    @functools.partial(
        pl.kernel, mesh=_sc_mesh(), out_type=out_ty,
        scratch_types=[
            pltpu.VMEM((1, K), jnp.int32),
            pltpu.VMEM((1, K), jnp.int32),
            pltpu.VMEM((1, K), jnp.int32),
            pltpu.VMEM((1, K), jnp.int32),
            pltpu.VMEM((K,), jnp.int32),
            pltpu.VMEM((K,), jnp.int32),
            pltpu.VMEM((K, D), jnp.float32),
            pltpu.VMEM((K, D), jnp.float32),
            pltpu.VMEM_SHARED((ACC_N, D), jnp.float32),
            pltpu.VMEM_SHARED((ACC_N, D), jnp.float32),
            pltpu.SemaphoreType.DMA,
            pltpu.SemaphoreType.DMA,
            pltpu.SemaphoreType.DMA,
            pltpu.SemaphoreType.DMA,
            pltpu.SemaphoreType.DMA,
            pltpu.SemaphoreType.DMA,
        ])
    def kern(table_h, src_h, dst_h, zero_h, outa_h, outb_h,
             srcra, srcrb, dstra, dstrb, dstja, dstjb, bufa, bufb,
             acc0, acc1, semga, semgb, semia, semib, semsa, semsb):
        c = lax.axis_index("c")
        s = lax.axis_index("s")
        lo = c * HALF
        sl = pl.ds(s * ACC_RPT, ACC_RPT)

        @pl.when(c == 0)
        def _():
            pltpu.sync_copy(zero_h.at[sl], acc0.at[sl])

        @pl.when(c == 1)
        def _():
            pltpu.sync_copy(zero_h.at[sl], acc1.at[sl])

        plsc.subcore_barrier()

        def scatter_start(buf, dstj, sem):
            @pl.when(c == 0)
            def _():
                pltpu.async_copy(buf, acc0.at[dstj], sem, add=True)

            @pl.when(c == 1)
            def _():
                pltpu.async_copy(buf, acc1.at[dstj], sem, add=True)

        def scatter_wait(buf, dstj, sem):
            @pl.when(c == 0)
            def _():
                pltpu.make_async_copy(buf, acc0.at[dstj], sem).wait()

            @pl.when(c == 1)
            def _():
                pltpu.make_async_copy(buf, acc1.at[dstj], sem).wait()

        pltpu.sync_copy(src_h.at[s, 0], srcra)
        pltpu.sync_copy(dst_h.at[s, 0], dstra)
        pltpu.async_copy(table_h.at[srcra.at[0]], bufa, semga)
        pltpu.async_copy(src_h.at[s, 1], srcrb, semib)
        pltpu.async_copy(dst_h.at[s, 1], dstrb, semib)

        @pl.loop(0, NCHUNK, step=2)
        def _(j):
            @pl.when(j > 0)
            def _():
                scatter_wait(bufb, dstjb, semsb)

            pltpu.make_async_copy(src_h.at[s, 0], srcrb, semib).wait()
            pltpu.make_async_copy(dst_h.at[s, 0], dstrb, semib).wait()
            pltpu.async_copy(table_h.at[srcrb.at[0]], bufb, semgb)

            @pl.when(j + 2 < NCHUNK)
            def _():
                pltpu.async_copy(src_h.at[s, j + 2], srcra, semia)

            pltpu.make_async_copy(table_h.at[srcra.at[0]], bufa, semga).wait()
            _stage_remap(dstra, dstja, lo, lo + HALF, lo, HALF)
            scatter_start(bufa, dstja, semsa)

            @pl.when(j + 2 < NCHUNK)
            def _():
                pltpu.async_copy(dst_h.at[s, j + 2], dstra, semia)

            pltpu.make_async_copy(table_h.at[srcrb.at[0]], bufb, semgb).wait()

            @pl.when(j + 2 < NCHUNK)
            def _():
                pltpu.make_async_copy(src_h.at[s, 0], srcra, semia).wait()
                pltpu.make_async_copy(dst_h.at[s, 0], dstra, semia).wait()
                scatter_wait(bufa, dstja, semsa)
                pltpu.async_copy(table_h.at[srcra.at[0]], bufa, semga)

            _stage_remap(dstrb, dstjb, lo, lo + HALF, lo, HALF)
            scatter_start(bufb, dstjb, semsb)

            @pl.when(j + 3 < NCHUNK)
            def _():
                pltpu.async_copy(src_h.at[s, j + 3], srcrb, semib)
                pltpu.async_copy(dst_h.at[s, j + 3], dstrb, semib)

        scatter_wait(bufa, dstja, semsa)
        scatter_wait(bufb, dstjb, semsb)
        plsc.subcore_barrier()

        @pl.when(c == 0)
        def _():
            pltpu.sync_copy(acc0.at[sl], outa_h.at[sl])

        @pl.when(c == 1)
        def _():
            pltpu.sync_copy(acc1.at[sl], outb_h.at[sl])

    return kern(table, src4, dst4, zeros_acc)


_BLK = 1000


def _tc_pre_body(deg_r, x_r, dis_r, xs_r):
    deg = deg_r[:, 0:1] + 1.0
    dis = lax.rsqrt(deg)
    disb = jnp.broadcast_to(dis, (_BLK, D))
    dis_r[...] = disb
    xs_r[...] = disb * x_r[...]


def _tc_pre(deg, x):
    return pl.pallas_call(
        _tc_pre_body,
        grid=(N_NODES // _BLK,),
        in_specs=[
            pl.BlockSpec((_BLK, D), lambda i: (i, 0)),
            pl.BlockSpec((_BLK, D), lambda i: (i, 0)),
        ],
        out_specs=[
            pl.BlockSpec((_BLK, D), lambda i: (i, 0)),
            pl.BlockSpec((_BLK, D), lambda i: (i, 0)),
        ],
        out_shape=[
            jax.ShapeDtypeStruct((N_NODES, D), jnp.float32),
            jax.ShapeDtypeStruct((N_NODES, D), jnp.float32),
        ],
    )(deg, x)


def _tc_mid_body(pa_r, xs_r, dis_r, w1_r, b1_r, w2_r, ts_r):
    p1 = dis_r[...] * (pa_r[...] + xs_r[...])
    h = jnp.dot(p1, w1_r[...], preferred_element_type=jnp.float32,
                precision=lax.Precision.HIGHEST) + b1_r[...]
    h = jnp.maximum(h, 0.0)
    t = jnp.dot(h, w2_r[...], preferred_element_type=jnp.float32,
                precision=lax.Precision.HIGHEST)
    ts_r[...] = dis_r[...] * t


def _tc_mid(pa, xs, dis, W1, b1, W2):
    d_hid = W1.shape[1]
    return pl.pallas_call(
        _tc_mid_body,
        grid=(N_NODES // _BLK,),
        in_specs=[
            pl.BlockSpec((_BLK, D), lambda i: (i, 0)),
            pl.BlockSpec((_BLK, D), lambda i: (i, 0)),
            pl.BlockSpec((_BLK, D), lambda i: (i, 0)),
            pl.BlockSpec((D, d_hid), lambda i: (0, 0)),
            pl.BlockSpec((1, d_hid), lambda i: (0, 0)),
            pl.BlockSpec((d_hid, D), lambda i: (0, 0)),
        ],
        out_specs=pl.BlockSpec((_BLK, D), lambda i: (i, 0)),
        out_shape=jax.ShapeDtypeStruct((N_NODES, D), jnp.float32),
    )(pa, xs, dis, W1, b1.reshape(1, d_hid), W2)


def _tc_post_body(pb_r, ts_r, dis_r, b2_r, out_r):
    out_r[...] = dis_r[...] * (pb_r[...] + ts_r[...]) + b2_r[...]


def _tc_post(pb, ts, dis, b2):
    return pl.pallas_call(
        _tc_post_body,
        grid=(N_NODES // _BLK,),
        in_specs=[
            pl.BlockSpec((_BLK, D), lambda i: (i, 0)),
            pl.BlockSpec((_BLK, D), lambda i: (i, 0)),
            pl.BlockSpec((_BLK, D), lambda i: (i, 0)),
            pl.BlockSpec((1, D), lambda i: (0, 0)),
        ],
        out_specs=pl.BlockSpec((_BLK, D), lambda i: (i, 0)),
        out_shape=jax.ShapeDtypeStruct((N_NODES, D), jnp.float32),
    )(pb, ts, dis, b2.reshape(1, D))


def _halves_to_full(pa_half, pb_half):
    return jnp.concatenate([pa_half[:HALF], pb_half[:HALF]], axis=0)


def kernel(x, edge_index, W1, b1, W2, b2):
    src = edge_index[0].astype(jnp.int32).reshape(NS, EPT)
    dst = edge_index[1].astype(jnp.int32).reshape(NS, EPT)
    pad = EPT_PAD - EPT
    src4 = jnp.pad(src, ((0, 0), (0, pad))).reshape(NS, NCHUNK, 1, K)
    dst4 = jnp.pad(dst, ((0, 0), (0, pad)),
                   constant_values=DUMMY_DST).reshape(NS, NCHUNK, 1, K)

    zeros_acc = jnp.zeros((ACC_N, D), jnp.float32)
    ones_blk = jnp.ones((K, D), jnp.float32)

    dega, degb = _sc_degree(dst4, zeros_acc, ones_blk)
    dis, xs = _tc_pre(_halves_to_full(dega, degb), x)
    paa, pab = _sc_propagate(xs, src4, dst4, zeros_acc)
    ts = _tc_mid(_halves_to_full(paa, pab), xs, dis, W1, b1, W2)
    pba, pbb = _sc_propagate(ts, src4, dst4, zeros_acc)
    return _tc_post(_halves_to_full(pba, pbb), ts, dis, b2)
